# baseline (speedup 1.0000x reference)
import functools

import jax
import jax.numpy as jnp
from jax import lax
from jax.experimental import pallas as pl
from jax.experimental.pallas import tpu as pltpu
from jax.experimental.pallas import tpu_sc as plsc

_B = 16384
_D = 64
_GAMMA = 12.0

_INFO = plsc.get_sparse_core_info()
_NC = _INFO.num_cores
_NS = _INFO.num_subcores
_NW = _NC * _NS
_L = _INFO.num_lanes
_BPW = _B // _NW
_CHUNK = 128
_NCHUNK = _BPW // _CHUNK

_mesh = plsc.VectorSubcoreMesh(core_axis_name="c", subcore_axis_name="s")


@functools.partial(
    pl.kernel,
    mesh=_mesh,
    out_type=jax.ShapeDtypeStruct((_B,), jnp.float32),
    compiler_params=pltpu.CompilerParams(
        needs_layout_passes=False, use_tc_tiling_on_sc=False
    ),
    scratch_types=[
        pltpu.VMEM((_NCHUNK, _CHUNK), jnp.int32),
        pltpu.VMEM((_NCHUNK, _CHUNK), jnp.int32),
        pltpu.VMEM((_NCHUNK, _CHUNK), jnp.int32),
        pltpu.VMEM((_BPW, _D), jnp.float32),
        pltpu.VMEM((_BPW, _D), jnp.float32),
        pltpu.VMEM((_BPW, _D), jnp.float32),
        pltpu.VMEM((_BPW,), jnp.float32),
        pltpu.SemaphoreType.DMA,
    ],
)
def _sc_score(hi_hbm, ri_hbm, ti_hbm, ent_hbm, rel_hbm, out_hbm,
              hi_v, ri_v, ti_v, h_rows, r_rows, t_rows, out_v, sem):
    wid = lax.axis_index("s") * _NC + lax.axis_index("c")
    row0 = wid * _NCHUNK

    pltpu.sync_copy(hi_hbm.at[pl.ds(row0, _NCHUNK)], hi_v)
    pltpu.sync_copy(ri_hbm.at[pl.ds(row0, _NCHUNK)], ri_v)
    pltpu.sync_copy(ti_hbm.at[pl.ds(row0, _NCHUNK)], ti_v)

    copies = []
    for j in range(_NCHUNK):
        dst = pl.ds(j * _CHUNK, _CHUNK)
        copies.append(pltpu.async_copy(ent_hbm.at[hi_v.at[j]], h_rows.at[dst], sem))
        copies.append(pltpu.async_copy(rel_hbm.at[ri_v.at[j]], r_rows.at[dst], sem))
        copies.append(pltpu.async_copy(ent_hbm.at[ti_v.at[j]], t_rows.at[dst], sem))
    for c in copies:
        c.wait()

    lane = lax.iota(jnp.int32, _L)

    def group_body(g, carry):
        rows = g * _L + lane
        acc = jnp.zeros((_L,), jnp.float32)
        for d in range(_D):
            col = jnp.full((_L,), d, jnp.int32)
            hv = plsc.load_gather(h_rows, [rows, col])
            rv = plsc.load_gather(r_rows, [rows, col])
            tv = plsc.load_gather(t_rows, [rows, col])
            acc = acc + jnp.abs(hv + rv - tv)
        plsc.store_scatter(out_v, [rows], _GAMMA - acc)
        return carry

    lax.fori_loop(0, _BPW // _L, group_body, 0)

    pltpu.sync_copy(out_v, out_hbm.at[pl.ds(wid * _BPW, _BPW)])


def kernel(sample, entity_embedding, relation_embedding):
    hi = sample[:, 0].reshape(_NW * _NCHUNK, _CHUNK)
    ri = sample[:, 1].reshape(_NW * _NCHUNK, _CHUNK)
    ti = sample[:, 2].reshape(_NW * _NCHUNK, _CHUNK)
    out = _sc_score(hi, ri, ti, entity_embedding, relation_embedding)
    return out.reshape(_B, 1)
